# Initial kernel scaffold; baseline (speedup 1.0000x reference)
import jax
import jax.numpy as jnp
from jax.experimental import pallas as pl


def kernel(depth, feat, ranks_depth, ranks_feat, ranks_bev, interval_starts, interval_lengths):
    raise NotImplementedError("write your pallas kernel here")


# speedup vs baseline: 3.4481x; 3.4481x over previous
import functools

import jax
import jax.numpy as jnp
from jax import lax
from jax.experimental import pallas as pl
from jax.experimental.pallas import tpu as pltpu
from jax.experimental.pallas import tpu_sc as plsc

C = 80
Z_OUT, H_OUT, W_OUT = 1, 256, 256
N_OUT = Z_OUT * H_OUT * W_OUT
R = 1024
NRANGES = N_OUT // R
K = 128
NW = 32
RANGES_PER_W = NRANGES // NW
BOUNDS_PAD = 80


def _sc_body(depth_hbm, feat_hbm, rd_hbm, rf_hbm, rb_hbm, bounds_hbm, out_hbm,
             bounds_v, rd_v, rf_v, rb_v, dg_v, fg_v, acc, sem0, sem1):
    wid = lax.axis_index("s") * 2 + lax.axis_index("c")
    pltpu.sync_copy(bounds_hbm, bounds_v)
    zero16 = jnp.zeros((16,), jnp.float32)
    for rg in range(RANGES_PER_W):
        j = wid * RANGES_PER_W + rg
        base = j * R
        sv = bounds_v[pl.ds(j, 16)]
        s = sv[0]
        e = sv[1]
        a = (s // 8) * 8
        nblk = (e - a + K - 1) // K

        def zrow(r, _):
            for cc in range(C // 16):
                acc[r, pl.ds(cc * 16, 16)] = zero16
            return 0
        lax.fori_loop(0, R, zrow, 0)

        def blk(ib, _):
            off = a + ib * K
            pltpu.sync_copy(rd_hbm.at[pl.ds(off, K)], rd_v)
            pltpu.sync_copy(rf_hbm.at[pl.ds(off, K)], rf_v)
            pltpu.sync_copy(rb_hbm.at[pl.ds(off, K)], rb_v)
            cp0 = pltpu.async_copy(depth_hbm.at[rd_v], dg_v, sem0)
            cp1 = pltpu.async_copy(feat_hbm.at[rf_v], fg_v, sem1)
            cp0.wait()
            cp1.wait()

            def grp(g, _):
                p16 = g * 16
                bv16 = rb_v[pl.ds(p16, 16)]
                wd16 = dg_v[pl.ds(p16, 16)]
                pg16 = off + p16 + lax.iota(jnp.int32, 16)
                ok = jnp.logical_and(pg16 < e, bv16 >= base)
                wv16 = jnp.where(ok, wd16, jnp.zeros((16,), jnp.float32))
                loc16 = jnp.clip(bv16 - base, 0, R - 1)
                for lane in range(16):
                    loc = loc16[lane]
                    wvec = jnp.full((16,), wv16[lane], jnp.float32)
                    p = p16 + lane
                    for cc in range(C // 16):
                        sl = pl.ds(cc * 16, 16)
                        acc[loc, sl] += wvec * fg_v[p, sl]
                return 0
            lax.fori_loop(0, K // 16, grp, 0)
            return 0
        lax.fori_loop(0, nblk, blk, 0)
        pltpu.sync_copy(acc, out_hbm.at[pl.ds(base, R)])


@jax.jit
def _bev_pool(depth_flat, feat_flat, rd_p, rf_p, rb_p, bounds):
    call = functools.partial(
        pl.kernel,
        out_type=jax.ShapeDtypeStruct((N_OUT, C), jnp.float32),
        mesh=plsc.VectorSubcoreMesh(core_axis_name="c", subcore_axis_name="s"),
        compiler_params=pltpu.CompilerParams(use_tc_tiling_on_sc=False),
        scratch_types=[
            pltpu.VMEM((BOUNDS_PAD,), jnp.int32),
            pltpu.VMEM((K,), jnp.int32),
            pltpu.VMEM((K,), jnp.int32),
            pltpu.VMEM((K,), jnp.int32),
            pltpu.VMEM((K,), jnp.float32),
            pltpu.VMEM((K, C), jnp.float32),
            pltpu.VMEM((R, C), jnp.float32),
            pltpu.SemaphoreType.DMA,
            pltpu.SemaphoreType.DMA,
        ],
    )(_sc_body)
    return call(depth_flat, feat_flat, rd_p, rf_p, rb_p, bounds)


def kernel(depth, feat, ranks_depth, ranks_feat, ranks_bev,
           interval_starts, interval_lengths):
    b = feat.shape[0]
    c = feat.shape[2]
    depth_flat = depth.reshape(-1)
    feat_flat = jnp.transpose(feat, (0, 1, 3, 4, 2)).reshape(-1, c)
    pad0 = jnp.zeros((K,), jnp.int32)
    rd_p = jnp.concatenate([ranks_depth, pad0])
    rf_p = jnp.concatenate([ranks_feat, pad0])
    rb_p = jnp.concatenate([ranks_bev, pad0])
    qs = jnp.arange(NRANGES + 1, dtype=jnp.int32) * R
    bounds = jnp.searchsorted(ranks_bev, qs).astype(jnp.int32)
    bounds = jnp.concatenate(
        [bounds, jnp.zeros((BOUNDS_PAD - (NRANGES + 1),), jnp.int32)])
    out = _bev_pool(depth_flat, feat_flat, rd_p, rf_p, rb_p, bounds)
    out = out.reshape(b, Z_OUT, H_OUT, W_OUT, c)
    return jnp.transpose(out, (0, 4, 1, 2, 3))


# speedup vs baseline: 5.1693x; 1.4992x over previous
import functools

import jax
import jax.numpy as jnp
from jax import lax
from jax.experimental import pallas as pl
from jax.experimental.pallas import tpu as pltpu
from jax.experimental.pallas import tpu_sc as plsc

C = 80
Z_OUT, H_OUT, W_OUT = 1, 256, 256
N_OUT = Z_OUT * H_OUT * W_OUT
R = 1024
NRANGES = N_OUT // R
K = 128
NW = 32
RANGES_PER_W = NRANGES // NW
BOUNDS_PAD = 80


def _sc_body(depth_hbm, feat_hbm, rd_hbm, rf_hbm, rb_hbm, bounds_hbm, out_hbm,
             bounds_v,
             rd0, rd1, rd2, rf0, rf1, rf2, rb0, rb1, rb2,
             dg0, dg1, fg0, fg1, acc,
             si0, si1, si2, sd0, sd1, sf0, sf1):
    n_pts = rd_hbm.shape[0] - K
    rd_v = (rd0, rd1, rd2)
    rf_v = (rf0, rf1, rf2)
    rb_v = (rb0, rb1, rb2)
    dg_v = (dg0, dg1)
    fg_v = (fg0, fg1)
    si = (si0, si1, si2)
    sd = (sd0, sd1)
    sf = (sf0, sf1)

    def issue_idx(off, i):
        pltpu.async_copy(rd_hbm.at[pl.ds(off, K)], rd_v[i], si[i])
        pltpu.async_copy(rf_hbm.at[pl.ds(off, K)], rf_v[i], si[i])
        pltpu.async_copy(rb_hbm.at[pl.ds(off, K)], rb_v[i], si[i])

    def wait_idx(i):
        pltpu.make_async_copy(rd_hbm.at[pl.ds(0, K)], rd_v[i], si[i]).wait()
        pltpu.make_async_copy(rf_hbm.at[pl.ds(0, K)], rf_v[i], si[i]).wait()
        pltpu.make_async_copy(rb_hbm.at[pl.ds(0, K)], rb_v[i], si[i]).wait()

    def issue_gather(i3, i2):
        pltpu.async_copy(depth_hbm.at[rd_v[i3]], dg_v[i2], sd[i2])
        pltpu.async_copy(feat_hbm.at[rf_v[i3]], fg_v[i2], sf[i2])

    def wait_gather(i3, i2):
        pltpu.make_async_copy(depth_hbm.at[rd_v[i3]], dg_v[i2], sd[i2]).wait()
        pltpu.make_async_copy(feat_hbm.at[rf_v[i3]], fg_v[i2], sf[i2]).wait()

    wid = lax.axis_index("s") * 2 + lax.axis_index("c")
    pltpu.sync_copy(bounds_hbm, bounds_v)
    zero16 = jnp.zeros((16,), jnp.float32)

    def range_body(rg, _):
        j = wid * RANGES_PER_W + rg
        base = j * R
        sv = bounds_v[pl.ds(j, 16)]
        s = sv[0]
        e = sv[1]
        a = (s // 8) * 8
        nblk = (e - a + K - 1) // K

        def zrow(r, _):
            for cc in range(C // 16):
                acc[r, pl.ds(cc * 16, 16)] = zero16
            return 0
        lax.fori_loop(0, R, zrow, 0)

        def compute(off, i3, i2):
            fg = fg_v[i2]

            def grp(g, _):
                p16 = g * 16
                bv16 = rb_v[i3][pl.ds(p16, 16)]
                wd16 = dg_v[i2][pl.ds(p16, 16)]
                pg16 = off + p16 + lax.iota(jnp.int32, 16)
                ok = jnp.logical_and(pg16 < e, bv16 >= base)
                wv16 = jnp.where(ok, wd16, jnp.zeros((16,), jnp.float32))
                loc16 = jnp.clip(bv16 - base, 0, R - 1)
                for lane in range(16):
                    loc = loc16[lane]
                    wvec = jnp.full((16,), wv16[lane], jnp.float32)
                    p = p16 + lane
                    for cc in range(C // 16):
                        sl = pl.ds(cc * 16, 16)
                        acc[loc, sl] += wvec * fg[p, sl]
                return 0
            lax.fori_loop(0, K // 16, grp, 0)

        issue_idx(jnp.minimum(a, n_pts), 0)
        issue_idx(jnp.minimum(a + K, n_pts), 1)
        wait_idx(0)
        issue_gather(0, 0)

        def round_body(tr, _):
            t0 = tr * 6
            for k in range(6):
                t = t0 + k
                issue_idx(jnp.minimum(a + (t + 2) * K, n_pts), (k + 2) % 3)
                wait_idx((k + 1) % 3)
                issue_gather((k + 1) % 3, (k + 1) % 2)
                wait_gather(k % 3, k % 2)
                compute(jnp.minimum(a + t * K, n_pts), k % 3, k % 2)
            return 0
        nround = (nblk + 5) // 6
        lax.fori_loop(0, nround, round_body, 0)
        wait_idx(1)
        wait_gather(0, 0)

        pltpu.sync_copy(acc, out_hbm.at[pl.ds(base, R)])
        return 0
    lax.fori_loop(0, RANGES_PER_W, range_body, 0)


@jax.jit
def _bev_pool(depth_flat, feat_flat, rd_p, rf_p, rb_p, bounds):
    call = functools.partial(
        pl.kernel,
        out_type=jax.ShapeDtypeStruct((N_OUT, C), jnp.float32),
        mesh=plsc.VectorSubcoreMesh(core_axis_name="c", subcore_axis_name="s"),
        compiler_params=pltpu.CompilerParams(use_tc_tiling_on_sc=False),
        scratch_types=[
            pltpu.VMEM((BOUNDS_PAD,), jnp.int32),
            pltpu.VMEM((K,), jnp.int32),
            pltpu.VMEM((K,), jnp.int32),
            pltpu.VMEM((K,), jnp.int32),
            pltpu.VMEM((K,), jnp.int32),
            pltpu.VMEM((K,), jnp.int32),
            pltpu.VMEM((K,), jnp.int32),
            pltpu.VMEM((K,), jnp.int32),
            pltpu.VMEM((K,), jnp.int32),
            pltpu.VMEM((K,), jnp.int32),
            pltpu.VMEM((K,), jnp.float32),
            pltpu.VMEM((K,), jnp.float32),
            pltpu.VMEM((K, C), jnp.float32),
            pltpu.VMEM((K, C), jnp.float32),
            pltpu.VMEM((R, C), jnp.float32),
            pltpu.SemaphoreType.DMA,
            pltpu.SemaphoreType.DMA,
            pltpu.SemaphoreType.DMA,
            pltpu.SemaphoreType.DMA,
            pltpu.SemaphoreType.DMA,
            pltpu.SemaphoreType.DMA,
            pltpu.SemaphoreType.DMA,
        ],
    )(_sc_body)
    return call(depth_flat, feat_flat, rd_p, rf_p, rb_p, bounds)


def kernel(depth, feat, ranks_depth, ranks_feat, ranks_bev,
           interval_starts, interval_lengths):
    b = feat.shape[0]
    c = feat.shape[2]
    depth_flat = depth.reshape(-1)
    feat_flat = jnp.transpose(feat, (0, 1, 3, 4, 2)).reshape(-1, c)
    pad0 = jnp.zeros((K,), jnp.int32)
    rd_p = jnp.concatenate([ranks_depth, pad0])
    rf_p = jnp.concatenate([ranks_feat, pad0])
    rb_p = jnp.concatenate([ranks_bev, pad0])
    qs = jnp.arange(NRANGES + 1, dtype=jnp.int32) * R
    bounds = jnp.searchsorted(ranks_bev, qs).astype(jnp.int32)
    bounds = jnp.concatenate(
        [bounds, jnp.zeros((BOUNDS_PAD - (NRANGES + 1),), jnp.int32)])
    out = _bev_pool(depth_flat, feat_flat, rd_p, rf_p, rb_p, bounds)
    out = out.reshape(b, Z_OUT, H_OUT, W_OUT, c)
    return jnp.transpose(out, (0, 4, 1, 2, 3))
